# Initial kernel scaffold; baseline (speedup 1.0000x reference)
import jax
import jax.numpy as jnp
from jax.experimental import pallas as pl


def kernel(xyz):
    raise NotImplementedError("write your pallas kernel here")


# speedup vs baseline: 7.8637x; 7.8637x over previous
import functools

import jax
import jax.numpy as jnp
from jax import lax
from jax.experimental import pallas as pl
from jax.experimental.pallas import tpu as pltpu
from jax.experimental.pallas import tpu_sc as plsc

NG = 512
K = 32


def _fps_body(x_ref, y_ref, z_ref, cx_ref, cy_ref, cz_ref):
    x = x_ref[...]
    y = y_ref[...]
    z = z_ref[...]
    B, N = x.shape
    iota_n = lax.broadcasted_iota(jnp.int32, (B, N), 1)
    iota_g = lax.broadcasted_iota(jnp.int32, (B, NG), 1)

    def step(g, carry):
        dist, far, cxs, cys, czs = carry
        sel = iota_n == far
        cx = jnp.sum(jnp.where(sel, x, 0.0), axis=1, keepdims=True)
        cy = jnp.sum(jnp.where(sel, y, 0.0), axis=1, keepdims=True)
        cz = jnp.sum(jnp.where(sel, z, 0.0), axis=1, keepdims=True)
        keep = iota_g == g
        cxs = jnp.where(keep, cx, cxs)
        cys = jnp.where(keep, cy, cys)
        czs = jnp.where(keep, cz, czs)
        d = (x - cx) ** 2 + (y - cy) ** 2 + (z - cz) ** 2
        dist = jnp.minimum(dist, d)
        mx = jnp.max(dist, axis=1, keepdims=True)
        far = jnp.min(jnp.where(dist == mx, iota_n, N), axis=1, keepdims=True)
        return dist, far, cxs, cys, czs

    init = (
        jnp.full((B, N), jnp.inf, jnp.float32),
        jnp.zeros((B, 1), jnp.int32),
        jnp.zeros((B, NG), jnp.float32),
        jnp.zeros((B, NG), jnp.float32),
        jnp.zeros((B, NG), jnp.float32),
    )
    _, _, cxs, cys, czs = lax.fori_loop(0, NG, step, init)
    cx_ref[...] = cxs
    cy_ref[...] = cys
    cz_ref[...] = czs


def _fps(x, y, z):
    B, N = x.shape
    out = jax.ShapeDtypeStruct((B, NG), jnp.float32)
    return pl.pallas_call(
        _fps_body,
        out_shape=(out, out, out),
    )(x, y, z)


_GB = 512


def _knn_body(x_ref, y_ref, z_ref, cxt_ref, cyt_ref, czt_ref, idx_ref):
    b = pl.program_id(0)
    B, N = x_ref.shape

    sub_iota = lax.broadcasted_iota(jnp.int32, (B, N), 0)
    brow = sub_iota == b
    px = jnp.sum(jnp.where(brow, x_ref[...], 0.0), axis=0, keepdims=True)
    py = jnp.sum(jnp.where(brow, y_ref[...], 0.0), axis=0, keepdims=True)
    pz = jnp.sum(jnp.where(brow, z_ref[...], 0.0), axis=0, keepdims=True)

    lane_b = lax.broadcasted_iota(jnp.int32, (_GB, B), 1)
    bcol = lane_b == b
    cx = jnp.sum(jnp.where(bcol, cxt_ref[...], 0.0), axis=1, keepdims=True)
    cy = jnp.sum(jnp.where(bcol, cyt_ref[...], 0.0), axis=1, keepdims=True)
    cz = jnp.sum(jnp.where(bcol, czt_ref[...], 0.0), axis=1, keepdims=True)

    c2 = cx * cx + cy * cy + cz * cz
    p2 = px * px + py * py + pz * pz
    cmat = jnp.concatenate([cx, cy, cz], axis=1)
    pmat = jnp.concatenate([px, py, pz], axis=0)
    dot = lax.dot_general(cmat, pmat, (((1,), (0,)), ((), ())),
                          preferred_element_type=jnp.float32)
    d2 = (c2 + p2) - 2.0 * dot

    iota_n = lax.broadcasted_iota(jnp.int32, (_GB, N), 1)
    iota_k = lax.broadcasted_iota(jnp.int32, (_GB, K), 1)

    def extract(t, carry):
        dprev, iprev, idxs = carry
        gt = (d2 > dprev) | ((d2 == dprev) & (iota_n > iprev))
        dm = jnp.min(jnp.where(gt, d2, jnp.inf), axis=1, keepdims=True)
        am = jnp.min(jnp.where((d2 == dm) & gt, iota_n, N), axis=1,
                     keepdims=True)
        idxs = jnp.where(iota_k == t, am, idxs)
        return dm, am, idxs

    init = (jnp.full((_GB, 1), -jnp.inf), jnp.full((_GB, 1), -1, jnp.int32),
            jnp.zeros((_GB, K), jnp.int32))
    _, _, idxs = lax.fori_loop(0, K, extract, init)
    idx_ref[0] = idxs


def _knn(x, y, z, cx, cy, cz):
    B, N = x.shape
    cxt = cx.T
    cyt = cy.T
    czt = cz.T
    full2 = lambda a: pl.BlockSpec(a.shape, lambda b: (0, 0))
    return pl.pallas_call(
        _knn_body,
        grid=(B,),
        in_specs=[
            full2(x), full2(y), full2(z),
            full2(cxt), full2(cyt), full2(czt),
        ],
        out_specs=pl.BlockSpec((1, _GB, K), lambda b: (b, 0, 0)),
        out_shape=jax.ShapeDtypeStruct((B, NG, K), jnp.int32),
    )(x, y, z, cxt, cyt, czt)


def _gather_sc(xf, yf, zf, idxf, cxf, cyf, czf, B, N):
    NW = 32
    WPB = NW // B
    GPW = NG // WPB
    mesh = plsc.VectorSubcoreMesh(core_axis_name="c", subcore_axis_name="s")

    @functools.partial(
        pl.kernel,
        mesh=mesh,
        compiler_params=pltpu.CompilerParams(needs_layout_passes=False),
        out_type=jax.ShapeDtypeStruct((B * NG * K * 3,), jnp.float32),
        scratch_types=[
            pltpu.VMEM((N,), jnp.float32),
            pltpu.VMEM((N,), jnp.float32),
            pltpu.VMEM((N,), jnp.float32),
            pltpu.VMEM((GPW * K,), jnp.int32),
            pltpu.VMEM((GPW,), jnp.float32),
            pltpu.VMEM((GPW,), jnp.float32),
            pltpu.VMEM((GPW,), jnp.float32),
            pltpu.VMEM((GPW * K * 3,), jnp.float32),
        ],
    )
    def body(xh, yh, zh, idxh, cxh, cyh, czh, out,
             xv, yv, zv, idxv, cxv, cyv, czv, nbv):
        wid = lax.axis_index("c") * 16 + lax.axis_index("s")
        b = wid // WPB
        h = wid % WPB
        pltpu.sync_copy(xh.at[pl.ds(pl.multiple_of(b * N, 8), N)], xv)
        pltpu.sync_copy(yh.at[pl.ds(pl.multiple_of(b * N, 8), N)], yv)
        pltpu.sync_copy(zh.at[pl.ds(pl.multiple_of(b * N, 8), N)], zv)
        goff = b * NG + h * GPW
        pltpu.sync_copy(
            idxh.at[pl.ds(pl.multiple_of(goff * K, 8), GPW * K)], idxv)
        pltpu.sync_copy(cxh.at[pl.ds(pl.multiple_of(goff, 8), GPW)], cxv)
        pltpu.sync_copy(cyh.at[pl.ds(pl.multiple_of(goff, 8), GPW)], cyv)
        pltpu.sync_copy(czh.at[pl.ds(pl.multiple_of(goff, 8), GPW)], czv)

        it = lax.broadcasted_iota(jnp.int32, (16,), 0)
        it3 = it * 3

        def group(g, _):
            gv = jnp.full((16,), g, jnp.int32)
            cgx = plsc.load_gather(cxv, [gv])
            cgy = plsc.load_gather(cyv, [gv])
            cgz = plsc.load_gather(czv, [gv])
            for kk in range(K // 16):
                iv = idxv[pl.ds(pl.multiple_of(g * K + kk * 16, 16), 16)]
                gx = plsc.load_gather(xv, [iv]) - cgx
                gy = plsc.load_gather(yv, [iv]) - cgy
                gz = plsc.load_gather(zv, [iv]) - cgz
                pos = (g * (K * 3) + kk * 48) + it3
                plsc.store_scatter(nbv, [pos], gx)
                plsc.store_scatter(nbv, [pos + 1], gy)
                plsc.store_scatter(nbv, [pos + 2], gz)
            return 0

        lax.fori_loop(0, GPW, group, 0)
        pltpu.sync_copy(
            nbv, out.at[pl.ds(pl.multiple_of(goff * K * 3, 8), GPW * K * 3)])

    return body(xf, yf, zf, idxf, cxf, cyf, czf)


def kernel(xyz):
    B, N, _ = xyz.shape
    x = xyz[:, :, 0]
    y = xyz[:, :, 1]
    z = xyz[:, :, 2]
    cx, cy, cz = _fps(x, y, z)
    idx = _knn(x, y, z, cx, cy, cz)
    nb_flat = _gather_sc(
        x.reshape(-1), y.reshape(-1), z.reshape(-1),
        idx.reshape(-1), cx.reshape(-1), cy.reshape(-1), cz.reshape(-1),
        B, N)
    neighborhood = nb_flat.reshape(B, NG, K, 3)
    center = jnp.stack([cx, cy, cz], axis=-1)
    return neighborhood, center


# speedup vs baseline: 15.8391x; 2.0142x over previous
import functools

import jax
import jax.numpy as jnp
from jax import lax
from jax.experimental import pallas as pl
from jax.experimental.pallas import tpu as pltpu
from jax.experimental.pallas import tpu_sc as plsc

NG = 512
K = 32


def _fps_body(x_ref, y_ref, z_ref, cx_ref, cy_ref, cz_ref):
    x = x_ref[...]
    y = y_ref[...]
    z = z_ref[...]
    B, N = x.shape
    iota_n = lax.broadcasted_iota(jnp.int32, (B, N), 1)
    iota_g = lax.broadcasted_iota(jnp.int32, (B, NG), 1)

    def step(g, carry):
        dist, far, cxs, cys, czs = carry
        sel = iota_n == far
        cx = jnp.sum(jnp.where(sel, x, 0.0), axis=1, keepdims=True)
        cy = jnp.sum(jnp.where(sel, y, 0.0), axis=1, keepdims=True)
        cz = jnp.sum(jnp.where(sel, z, 0.0), axis=1, keepdims=True)
        keep = iota_g == g
        cxs = jnp.where(keep, cx, cxs)
        cys = jnp.where(keep, cy, cys)
        czs = jnp.where(keep, cz, czs)
        d = (x - cx) ** 2 + (y - cy) ** 2 + (z - cz) ** 2
        dist = jnp.minimum(dist, d)
        mx = jnp.max(dist, axis=1, keepdims=True)
        far = jnp.min(jnp.where(dist == mx, iota_n, N), axis=1, keepdims=True)
        return dist, far, cxs, cys, czs

    init = (
        jnp.full((B, N), jnp.inf, jnp.float32),
        jnp.zeros((B, 1), jnp.int32),
        jnp.zeros((B, NG), jnp.float32),
        jnp.zeros((B, NG), jnp.float32),
        jnp.zeros((B, NG), jnp.float32),
    )
    _, _, cxs, cys, czs = lax.fori_loop(0, NG, step, init)
    cx_ref[...] = cxs
    cy_ref[...] = cys
    cz_ref[...] = czs


def _fps(x, y, z):
    B, N = x.shape
    out = jax.ShapeDtypeStruct((B, NG), jnp.float32)
    return pl.pallas_call(
        _fps_body,
        out_shape=(out, out, out),
    )(x, y, z)


_GB = 512


def _knn_body(x_ref, y_ref, z_ref, cxt_ref, cyt_ref, czt_ref, idx_ref):
    b = pl.program_id(0)
    B, N = x_ref.shape

    sub_iota = lax.broadcasted_iota(jnp.int32, (B, N), 0)
    brow = sub_iota == b
    px = jnp.sum(jnp.where(brow, x_ref[...], 0.0), axis=0, keepdims=True)
    py = jnp.sum(jnp.where(brow, y_ref[...], 0.0), axis=0, keepdims=True)
    pz = jnp.sum(jnp.where(brow, z_ref[...], 0.0), axis=0, keepdims=True)

    lane_b = lax.broadcasted_iota(jnp.int32, (_GB, B), 1)
    bcol = lane_b == b
    cx = jnp.sum(jnp.where(bcol, cxt_ref[...], 0.0), axis=1, keepdims=True)
    cy = jnp.sum(jnp.where(bcol, cyt_ref[...], 0.0), axis=1, keepdims=True)
    cz = jnp.sum(jnp.where(bcol, czt_ref[...], 0.0), axis=1, keepdims=True)

    c2 = cx * cx + cy * cy + cz * cz
    p2 = px * px + py * py + pz * pz
    cmat = jnp.concatenate([cx, cy, cz], axis=1)
    pmat = jnp.concatenate([px, py, pz], axis=0)
    dot = lax.dot_general(cmat, pmat, (((1,), (0,)), ((), ())),
                          preferred_element_type=jnp.float32)
    d2 = (c2 + p2) - 2.0 * dot

    iota_n = lax.broadcasted_iota(jnp.int32, (_GB, N), 1)
    iota_k = lax.broadcasted_iota(jnp.int32, (_GB, K), 1)

    def extract(t, carry):
        dprev, iprev, m, idxs = carry
        lex = (d2 > dprev) | ((d2 == dprev) & (iota_n > iprev))
        eqm = lex & (d2 == m)
        am = jnp.min(jnp.where(eqm, iota_n, N), axis=1, keepdims=True)
        cnt = jnp.sum(jnp.where(eqm, 1.0, 0.0), axis=1, keepdims=True)
        strict = jnp.min(jnp.where(d2 > m, d2, jnp.inf), axis=1,
                         keepdims=True)
        mnext = jnp.where(cnt >= 2.0, m, strict)
        idxs = jnp.where(iota_k == t, am, idxs)
        return m, am, mnext, idxs

    m0 = jnp.min(d2, axis=1, keepdims=True)
    init = (jnp.full((_GB, 1), -jnp.inf), jnp.full((_GB, 1), -1, jnp.int32),
            m0, jnp.zeros((_GB, K), jnp.int32))
    _, _, _, idxs = lax.fori_loop(0, K, extract, init)
    idx_ref[0] = idxs


def _knn(x, y, z, cx, cy, cz):
    B, N = x.shape
    cxt = cx.T
    cyt = cy.T
    czt = cz.T
    full2 = lambda a: pl.BlockSpec(a.shape, lambda b: (0, 0))
    return pl.pallas_call(
        _knn_body,
        grid=(B,),
        in_specs=[
            full2(x), full2(y), full2(z),
            full2(cxt), full2(cyt), full2(czt),
        ],
        out_specs=pl.BlockSpec((1, _GB, K), lambda b: (b, 0, 0)),
        out_shape=jax.ShapeDtypeStruct((B, NG, K), jnp.int32),
    )(x, y, z, cxt, cyt, czt)


def _gather_sc(xf, yf, zf, idxf, cxf, cyf, czf, B, N):
    NW = 32
    WPB = NW // B
    GPW = NG // WPB
    mesh = plsc.VectorSubcoreMesh(core_axis_name="c", subcore_axis_name="s")

    @functools.partial(
        pl.kernel,
        mesh=mesh,
        compiler_params=pltpu.CompilerParams(needs_layout_passes=False),
        out_type=jax.ShapeDtypeStruct((B * NG * K * 3,), jnp.float32),
        scratch_types=[
            pltpu.VMEM((N,), jnp.float32),
            pltpu.VMEM((N,), jnp.float32),
            pltpu.VMEM((N,), jnp.float32),
            pltpu.VMEM((GPW * K,), jnp.int32),
            pltpu.VMEM((GPW,), jnp.float32),
            pltpu.VMEM((GPW,), jnp.float32),
            pltpu.VMEM((GPW,), jnp.float32),
            pltpu.VMEM((GPW * K * 3,), jnp.float32),
        ],
    )
    def body(xh, yh, zh, idxh, cxh, cyh, czh, out,
             xv, yv, zv, idxv, cxv, cyv, czv, nbv):
        wid = lax.axis_index("c") * 16 + lax.axis_index("s")
        b = wid // WPB
        h = wid % WPB
        pltpu.sync_copy(xh.at[pl.ds(pl.multiple_of(b * N, 8), N)], xv)
        pltpu.sync_copy(yh.at[pl.ds(pl.multiple_of(b * N, 8), N)], yv)
        pltpu.sync_copy(zh.at[pl.ds(pl.multiple_of(b * N, 8), N)], zv)
        goff = b * NG + h * GPW
        pltpu.sync_copy(
            idxh.at[pl.ds(pl.multiple_of(goff * K, 8), GPW * K)], idxv)
        pltpu.sync_copy(cxh.at[pl.ds(pl.multiple_of(goff, 8), GPW)], cxv)
        pltpu.sync_copy(cyh.at[pl.ds(pl.multiple_of(goff, 8), GPW)], cyv)
        pltpu.sync_copy(czh.at[pl.ds(pl.multiple_of(goff, 8), GPW)], czv)

        it = lax.broadcasted_iota(jnp.int32, (16,), 0)
        it3 = it * 3

        def group(g, _):
            gv = jnp.full((16,), g, jnp.int32)
            cgx = plsc.load_gather(cxv, [gv])
            cgy = plsc.load_gather(cyv, [gv])
            cgz = plsc.load_gather(czv, [gv])
            for kk in range(K // 16):
                iv = idxv[pl.ds(pl.multiple_of(g * K + kk * 16, 16), 16)]
                gx = plsc.load_gather(xv, [iv]) - cgx
                gy = plsc.load_gather(yv, [iv]) - cgy
                gz = plsc.load_gather(zv, [iv]) - cgz
                pos = (g * (K * 3) + kk * 48) + it3
                plsc.store_scatter(nbv, [pos], gx)
                plsc.store_scatter(nbv, [pos + 1], gy)
                plsc.store_scatter(nbv, [pos + 2], gz)
            return 0

        lax.fori_loop(0, GPW, group, 0)
        pltpu.sync_copy(
            nbv, out.at[pl.ds(pl.multiple_of(goff * K * 3, 8), GPW * K * 3)])

    return body(xf, yf, zf, idxf, cxf, cyf, czf)


def kernel(xyz):
    B, N, _ = xyz.shape
    x = xyz[:, :, 0]
    y = xyz[:, :, 1]
    z = xyz[:, :, 2]
    cx, cy, cz = _fps(x, y, z)
    idx = _knn(x, y, z, cx, cy, cz)
    nb_flat = _gather_sc(
        x.reshape(-1), y.reshape(-1), z.reshape(-1),
        idx.reshape(-1), cx.reshape(-1), cy.reshape(-1), cz.reshape(-1),
        B, N)
    neighborhood = nb_flat.reshape(B, NG, K, 3)
    center = jnp.stack([cx, cy, cz], axis=-1)
    return neighborhood, center


# speedup vs baseline: 17.2747x; 1.0906x over previous
import functools

import numpy as np
import jax
import jax.numpy as jnp
from jax import lax
from jax.experimental import pallas as pl
from jax.experimental.pallas import tpu as pltpu
from jax.experimental.pallas import tpu_sc as plsc

NG = 512
K = 32
CW = 16
NC = 512
SEL = 32
CAND = SEL * CW

_PERM = (CW * (np.arange(NC * CW) % NC) + np.arange(NC * CW) // NC).astype(
    np.int32)


def _fps_body(x_ref, y_ref, z_ref, cx_ref, cy_ref, cz_ref):
    x = x_ref[...]
    y = y_ref[...]
    z = z_ref[...]
    B, N = x.shape
    iota_n = lax.broadcasted_iota(jnp.int32, (B, N), 1)
    iota_g = lax.broadcasted_iota(jnp.int32, (B, NG), 1)

    def step(g, carry):
        dist, far, cxs, cys, czs = carry
        sel = iota_n == far
        cx = jnp.sum(jnp.where(sel, x, 0.0), axis=1, keepdims=True)
        cy = jnp.sum(jnp.where(sel, y, 0.0), axis=1, keepdims=True)
        cz = jnp.sum(jnp.where(sel, z, 0.0), axis=1, keepdims=True)
        keep = iota_g == g
        cxs = jnp.where(keep, cx, cxs)
        cys = jnp.where(keep, cy, cys)
        czs = jnp.where(keep, cz, czs)
        d = (x - cx) ** 2 + (y - cy) ** 2 + (z - cz) ** 2
        dist = jnp.minimum(dist, d)
        mx = jnp.max(dist, axis=1, keepdims=True)
        far = jnp.min(jnp.where(dist == mx, iota_n, N), axis=1, keepdims=True)
        return dist, far, cxs, cys, czs

    init = (
        jnp.full((B, N), jnp.inf, jnp.float32),
        jnp.zeros((B, 1), jnp.int32),
        jnp.zeros((B, NG), jnp.float32),
        jnp.zeros((B, NG), jnp.float32),
        jnp.zeros((B, NG), jnp.float32),
    )
    _, _, cxs, cys, czs = lax.fori_loop(0, NG, step, init)
    cx_ref[...] = cxs
    cy_ref[...] = cys
    cz_ref[...] = czs


def _fps(x, y, z):
    B, N = x.shape
    out = jax.ShapeDtypeStruct((B, NG), jnp.float32)
    return pl.pallas_call(
        _fps_body,
        out_shape=(out, out, out),
    )(x, y, z)


_GJ = 256


def _row_select(ref, b):
    B, N = ref.shape
    sub_iota = lax.broadcasted_iota(jnp.int32, (B, N), 0)
    return jnp.sum(jnp.where(sub_iota == b, ref[...], 0.0), axis=0,
                   keepdims=True)


def _knn2_body(x_ref, y_ref, z_ref, xq_ref, yq_ref, zq_ref,
               cxt_ref, cyt_ref, czt_ref, d2_ref, chsel_ref):
    b = pl.program_id(0)
    j = pl.program_id(1)
    B, N = x_ref.shape

    px = _row_select(x_ref, b)
    py = _row_select(y_ref, b)
    pz = _row_select(z_ref, b)
    pqx = _row_select(xq_ref, b)
    pqy = _row_select(yq_ref, b)
    pqz = _row_select(zq_ref, b)

    lane_b = lax.broadcasted_iota(jnp.int32, (_GJ, B), 1)
    bcol = lane_b == b
    sl = pl.ds(j * _GJ, _GJ)
    cx = jnp.sum(jnp.where(bcol, cxt_ref[sl, :], 0.0), axis=1, keepdims=True)
    cy = jnp.sum(jnp.where(bcol, cyt_ref[sl, :], 0.0), axis=1, keepdims=True)
    cz = jnp.sum(jnp.where(bcol, czt_ref[sl, :], 0.0), axis=1, keepdims=True)

    c2 = cx * cx + cy * cy + cz * cz
    p2 = px * px + py * py + pz * pz
    p2q = pqx * pqx + pqy * pqy + pqz * pqz
    cmat = jnp.concatenate([cx, cy, cz], axis=1)
    pmat = jnp.concatenate([px, py, pz], axis=0)
    pmatq = jnp.concatenate([pqx, pqy, pqz], axis=0)
    dot = lax.dot_general(cmat, pmat, (((1,), (0,)), ((), ())),
                          preferred_element_type=jnp.float32)
    d2 = (c2 + p2) - 2.0 * dot
    d2_ref[0] = d2
    dotq = lax.dot_general(cmat, pmatq, (((1,), (0,)), ((), ())),
                           preferred_element_type=jnp.float32)
    d2q = (c2 + p2q) - 2.0 * dotq

    M = d2q[:, 0:NC]
    for s in range(1, CW):
        M = jnp.minimum(M, d2q[:, s * NC:(s + 1) * NC])

    ciota = lax.broadcasted_iota(jnp.int32, (_GJ, NC), 1)
    iota_s = lax.broadcasted_iota(jnp.int32, (_GJ, SEL), 1)

    def pick(t, carry):
        Mw, sel = carry
        mm = jnp.min(Mw, axis=1, keepdims=True)
        cc = jnp.min(jnp.where(Mw == mm, ciota, NC), axis=1, keepdims=True)
        sel = jnp.where(iota_s == t, cc, sel)
        Mw = jnp.where(ciota == cc, jnp.inf, Mw)
        return Mw, sel

    _, sel = lax.fori_loop(0, SEL, pick,
                           (M, jnp.zeros((_GJ, SEL), jnp.int32)))

    def sort_step(t, carry):
        prev, out = carry
        nid = jnp.min(jnp.where(sel > prev, sel, NC), axis=1, keepdims=True)
        out = jnp.where(iota_s == t, nid, out)
        return nid, out

    _, chsorted = lax.fori_loop(
        0, SEL, sort_step,
        (jnp.full((_GJ, 1), -1, jnp.int32), jnp.zeros((_GJ, SEL), jnp.int32)))
    chsel_ref[0] = chsorted


def _knn2(x, y, z, xq, yq, zq, cx, cy, cz):
    B, N = x.shape
    cxt, cyt, czt = cx.T, cy.T, cz.T
    full2 = lambda a: pl.BlockSpec(a.shape, lambda b, j: (0, 0))
    return pl.pallas_call(
        _knn2_body,
        grid=(B, NG // _GJ),
        in_specs=[
            full2(x), full2(y), full2(z),
            full2(xq), full2(yq), full2(zq),
            full2(cxt), full2(cyt), full2(czt),
        ],
        out_specs=[
            pl.BlockSpec((1, _GJ, N), lambda b, j: (b, j, 0)),
            pl.BlockSpec((1, _GJ, SEL), lambda b, j: (b, j, 0)),
        ],
        out_shape=[
            jax.ShapeDtypeStruct((B, NG, N), jnp.float32),
            jax.ShapeDtypeStruct((B, NG, SEL), jnp.int32),
        ],
    )(x, y, z, xq, yq, zq, cxt, cyt, czt)


def _candgather_sc(d2tab, chself, B):
    NW = 32
    GPW = B * NG // NW
    RW = 128
    NF = 32
    HROWS = NF * RW
    NH = GPW * SEL // HROWS
    mesh = plsc.VectorSubcoreMesh(core_axis_name="c", subcore_axis_name="s")

    @functools.partial(
        pl.kernel,
        mesh=mesh,
        compiler_params=pltpu.CompilerParams(needs_layout_passes=False,
                                             use_tc_tiling_on_sc=False),
        out_type=jax.ShapeDtypeStruct((B * NG * SEL, CW), jnp.float32),
        scratch_types=[
            pltpu.VMEM((GPW * SEL,), jnp.int32),
            pltpu.VMEM((GPW * SEL,), jnp.int32),
            pltpu.VMEM((HROWS, CW), jnp.float32),
            pltpu.SemaphoreType.DMA,
        ],
    )
    def body(d2h, chh, out, chv, idxv, candv, sem):
        wid = lax.axis_index("c") * 16 + lax.axis_index("s")
        pltpu.sync_copy(
            chh.at[pl.ds(pl.multiple_of(wid * GPW * SEL, 8), GPW * SEL)], chv)

        for g in range(GPW):
            base = (wid * GPW + g) * NC
            for q in range(SEL // 16):
                sl = pl.ds(pl.multiple_of(g * SEL + q * 16, 16), 16)
                idxv[sl] = chv[sl] + base

        for half in range(NH):
            hoff = half * HROWS
            for f in range(NF):
                pltpu.make_async_copy(
                    d2h.at[idxv.at[pl.ds(
                        pl.multiple_of(hoff + f * RW, 8), RW)]],
                    candv.at[pl.ds(f * RW, RW)], sem).start()
            for f in range(NF):
                pltpu.make_async_copy(
                    d2h.at[idxv.at[pl.ds(
                        pl.multiple_of(hoff + f * RW, 8), RW)]],
                    candv.at[pl.ds(f * RW, RW)], sem).wait()
            roff = wid * GPW * SEL + hoff
            pltpu.sync_copy(candv,
                            out.at[pl.ds(pl.multiple_of(roff, 8), HROWS)])

    return body(d2tab, chself)


def _extract_body(cand_ref, chsel_ref, idx_ref):
    cd = cand_ref[0]
    ch = chsel_ref[0]
    iota_c = lax.broadcasted_iota(jnp.int32, (NG, CAND), 1)
    iota_s = lax.broadcasted_iota(jnp.int32, (NG, SEL), 1)
    iota_k = lax.broadcasted_iota(jnp.int32, (NG, K), 1)

    def extract(t, carry):
        Dw, idxs = carry
        m = jnp.min(Dw, axis=1, keepdims=True)
        am = jnp.min(jnp.where(Dw == m, iota_c, CAND), axis=1, keepdims=True)
        slot = am // CW
        within = am - slot * CW
        oc = jnp.sum(jnp.where(iota_s == slot, ch, 0), axis=1, keepdims=True)
        orig = oc * CW + within
        idxs = jnp.where(iota_k == t, orig, idxs)
        Dw = jnp.where(iota_c == am, jnp.inf, Dw)
        return Dw, idxs

    _, idxs = lax.fori_loop(0, K, extract,
                            (cd, jnp.zeros((NG, K), jnp.int32)))
    idx_ref[0] = idxs


def _extract(cand, chsel):
    B = cand.shape[0]
    return pl.pallas_call(
        _extract_body,
        grid=(B,),
        in_specs=[
            pl.BlockSpec((1, NG, CAND), lambda b: (b, 0, 0)),
            pl.BlockSpec((1, NG, SEL), lambda b: (b, 0, 0)),
        ],
        out_specs=pl.BlockSpec((1, NG, K), lambda b: (b, 0, 0)),
        out_shape=jax.ShapeDtypeStruct((B, NG, K), jnp.int32),
    )(cand, chsel)


def _gather_sc(xf, yf, zf, idxf, cxf, cyf, czf, B, N):
    NW = 32
    WPB = NW // B
    GPW = NG // WPB
    mesh = plsc.VectorSubcoreMesh(core_axis_name="c", subcore_axis_name="s")

    @functools.partial(
        pl.kernel,
        mesh=mesh,
        compiler_params=pltpu.CompilerParams(needs_layout_passes=False),
        out_type=jax.ShapeDtypeStruct((B * NG * K * 3,), jnp.float32),
        scratch_types=[
            pltpu.VMEM((N,), jnp.float32),
            pltpu.VMEM((N,), jnp.float32),
            pltpu.VMEM((N,), jnp.float32),
            pltpu.VMEM((GPW * K,), jnp.int32),
            pltpu.VMEM((GPW,), jnp.float32),
            pltpu.VMEM((GPW,), jnp.float32),
            pltpu.VMEM((GPW,), jnp.float32),
            pltpu.VMEM((GPW * K * 3,), jnp.float32),
        ],
    )
    def body(xh, yh, zh, idxh, cxh, cyh, czh, out,
             xv, yv, zv, idxv, cxv, cyv, czv, nbv):
        wid = lax.axis_index("c") * 16 + lax.axis_index("s")
        b = wid // WPB
        h = wid % WPB
        pltpu.sync_copy(xh.at[pl.ds(pl.multiple_of(b * N, 8), N)], xv)
        pltpu.sync_copy(yh.at[pl.ds(pl.multiple_of(b * N, 8), N)], yv)
        pltpu.sync_copy(zh.at[pl.ds(pl.multiple_of(b * N, 8), N)], zv)
        goff = b * NG + h * GPW
        pltpu.sync_copy(
            idxh.at[pl.ds(pl.multiple_of(goff * K, 8), GPW * K)], idxv)
        pltpu.sync_copy(cxh.at[pl.ds(pl.multiple_of(goff, 8), GPW)], cxv)
        pltpu.sync_copy(cyh.at[pl.ds(pl.multiple_of(goff, 8), GPW)], cyv)
        pltpu.sync_copy(czh.at[pl.ds(pl.multiple_of(goff, 8), GPW)], czv)

        it = lax.broadcasted_iota(jnp.int32, (16,), 0)
        it3 = it * 3

        def group(g, _):
            gv = jnp.full((16,), g, jnp.int32)
            cgx = plsc.load_gather(cxv, [gv])
            cgy = plsc.load_gather(cyv, [gv])
            cgz = plsc.load_gather(czv, [gv])
            for kk in range(K // 16):
                iv = idxv[pl.ds(pl.multiple_of(g * K + kk * 16, 16), 16)]
                gx = plsc.load_gather(xv, [iv]) - cgx
                gy = plsc.load_gather(yv, [iv]) - cgy
                gz = plsc.load_gather(zv, [iv]) - cgz
                pos = (g * (K * 3) + kk * 48) + it3
                plsc.store_scatter(nbv, [pos], gx)
                plsc.store_scatter(nbv, [pos + 1], gy)
                plsc.store_scatter(nbv, [pos + 2], gz)
            return 0

        lax.fori_loop(0, GPW, group, 0)
        pltpu.sync_copy(
            nbv, out.at[pl.ds(pl.multiple_of(goff * K * 3, 8), GPW * K * 3)])

    return body(xf, yf, zf, idxf, cxf, cyf, czf)


def kernel(xyz):
    B, N, _ = xyz.shape
    x = xyz[:, :, 0]
    y = xyz[:, :, 1]
    z = xyz[:, :, 2]
    tq = lambda a: a.reshape(B, NC, CW).swapaxes(1, 2).reshape(B, N)
    xq = tq(x)
    yq = tq(y)
    zq = tq(z)
    cx, cy, cz = _fps(x, y, z)
    d2tab, chsel = _knn2(x, y, z, xq, yq, zq, cx, cy, cz)
    cand = _candgather_sc(d2tab.reshape(B * NG * NC, CW),
                          chsel.reshape(-1), B)
    idx = _extract(cand.reshape(B, NG, CAND), chsel)
    nb_flat = _gather_sc(
        x.reshape(-1), y.reshape(-1), z.reshape(-1),
        idx.reshape(-1), cx.reshape(-1), cy.reshape(-1), cz.reshape(-1),
        B, N)
    neighborhood = nb_flat.reshape(B, NG, K, 3)
    center = jnp.stack([cx, cy, cz], axis=-1)
    return neighborhood, center
